# baseline (speedup 1.0000x reference)
import functools

import jax
import jax.numpy as jnp
from jax import lax
from jax.experimental import pallas as pl
from jax.experimental.pallas import tpu as pltpu
from jax.experimental.pallas import tpu_sc as plsc

N = 10000
E = 640000
D_IN = 128
H = 64
G = 64
NB = 3

NC = 2
NS = 16
NW = NC * NS

CHUNK = 128
CH = 158
EPW = CH * CHUNK
E_PAD = NW * EPW
DUMMY = N
N_ACC = 10112
ZPT = N_ACC // NS
LAST = N - (NS - 1) * ZPT

BP = 2000
GRID = N // BP

_MESH = plsc.VectorSubcoreMesh(core_axis_name="c", subcore_axis_name="s")


def _deg_body(dst_hbm, ones_hbm, zero_hbm, out_hbm, dst_v, ones_v, acc_sh):
    c = lax.axis_index("c")
    s = lax.axis_index("s")
    w = c * NS + s
    pltpu.sync_copy(dst_hbm.at[w], dst_v)
    pltpu.sync_copy(ones_hbm, ones_v)
    pltpu.sync_copy(zero_hbm.at[pl.ds(s * ZPT, ZPT)], acc_sh.at[pl.ds(s * ZPT, ZPT)])
    plsc.subcore_barrier()

    def body(j, carry):
        pltpu.sync_copy(ones_v, acc_sh.at[dst_v.at[j]], add=True)
        return carry

    lax.fori_loop(0, CH, body, 0)
    plsc.subcore_barrier()

    @pl.when(s < NS - 1)
    def _():
        pltpu.sync_copy(acc_sh.at[pl.ds(s * ZPT, ZPT)],
                        out_hbm.at[c, pl.ds(s * ZPT, ZPT)])

    @pl.when(s == NS - 1)
    def _():
        pltpu.sync_copy(acc_sh.at[pl.ds((NS - 1) * ZPT, LAST)],
                        out_hbm.at[c, pl.ds((NS - 1) * ZPT, LAST)])


_deg_call = pl.kernel(
    _deg_body,
    out_type=jax.ShapeDtypeStruct((NC, N, 16), jnp.float32),
    mesh=_MESH,
    compiler_params=pltpu.CompilerParams(use_tc_tiling_on_sc=False),
    scratch_types=[
        pltpu.VMEM((CH, CHUNK), jnp.int32),
        pltpu.VMEM((CHUNK, 16), jnp.float32),
        pltpu.VMEM_SHARED((N_ACC, 16), jnp.float32),
    ],
)


def _agg_body(h_hbm, src_hbm, dst_hbm, zero_hbm, out_hbm,
              src_v, dst_v, rows0, rows1, acc_sh, g0, g1, s0, s1):
    c = lax.axis_index("c")
    s = lax.axis_index("s")
    w = c * NS + s
    pltpu.sync_copy(src_hbm.at[w], src_v)
    pltpu.sync_copy(dst_hbm.at[w], dst_v)
    pltpu.sync_copy(zero_hbm.at[pl.ds(s * ZPT, ZPT)], acc_sh.at[pl.ds(s * ZPT, ZPT)])
    plsc.subcore_barrier()

    pltpu.async_copy(h_hbm.at[src_v.at[0]], rows0, g0)

    def body(u, carry):
        j0 = 2 * u
        j1 = 2 * u + 1
        pltpu.make_async_copy(h_hbm.at[src_v.at[j0]], rows0, g0).wait()
        pltpu.async_copy(h_hbm.at[src_v.at[j1]], rows1, g1)
        pltpu.async_copy(rows0, acc_sh.at[dst_v.at[j0]], s0, add=True)
        pltpu.make_async_copy(h_hbm.at[src_v.at[j1]], rows1, g1).wait()
        pltpu.make_async_copy(rows0, acc_sh.at[dst_v.at[j0]], s0).wait()
        pltpu.async_copy(h_hbm.at[src_v.at[jnp.minimum(j1 + 1, CH - 1)]], rows0, g0)
        pltpu.async_copy(rows1, acc_sh.at[dst_v.at[j1]], s1, add=True)
        pltpu.make_async_copy(rows1, acc_sh.at[dst_v.at[j1]], s1).wait()
        return carry

    lax.fori_loop(0, CH // 2, body, 0)
    pltpu.make_async_copy(h_hbm.at[src_v.at[CH - 1]], rows0, g0).wait()
    plsc.subcore_barrier()

    @pl.when(s < NS - 1)
    def _():
        pltpu.sync_copy(acc_sh.at[pl.ds(s * ZPT, ZPT)],
                        out_hbm.at[c, pl.ds(s * ZPT, ZPT)])

    @pl.when(s == NS - 1)
    def _():
        pltpu.sync_copy(acc_sh.at[pl.ds((NS - 1) * ZPT, LAST)],
                        out_hbm.at[c, pl.ds((NS - 1) * ZPT, LAST)])


_agg_call = pl.kernel(
    _agg_body,
    out_type=jax.ShapeDtypeStruct((NC, N, H), jnp.float32),
    mesh=_MESH,
    compiler_params=pltpu.CompilerParams(use_tc_tiling_on_sc=False),
    scratch_types=[
        pltpu.VMEM((CH, CHUNK), jnp.int32),
        pltpu.VMEM((CH, CHUNK), jnp.int32),
        pltpu.VMEM((CHUNK, H), jnp.float32),
        pltpu.VMEM((CHUNK, H), jnp.float32),
        pltpu.VMEM_SHARED((N_ACC, H), jnp.float32),
        pltpu.SemaphoreType.DMA,
        pltpu.SemaphoreType.DMA,
        pltpu.SemaphoreType.DMA,
        pltpu.SemaphoreType.DMA,
    ],
)


def _ln(x, g, b, eps=1e-5):
    m = jnp.mean(x, axis=-1, keepdims=True)
    v = jnp.mean((x - m) * (x - m), axis=-1, keepdims=True)
    return (x - m) / jnp.sqrt(v + eps) * g + b


def _proj_body(x_ref, w_ref, b_ref, h_ref):
    h_ref[...] = (jnp.dot(x_ref[...], w_ref[...],
                          preferred_element_type=jnp.float32) + b_ref[...])


_proj_call = pl.pallas_call(
    _proj_body,
    grid=(GRID,),
    in_specs=[
        pl.BlockSpec((BP, D_IN), lambda i: (i, 0)),
        pl.BlockSpec((D_IN, H), lambda i: (0, 0)),
        pl.BlockSpec((1, H), lambda i: (0, 0)),
    ],
    out_specs=pl.BlockSpec((BP, H), lambda i: (i, 0)),
    out_shape=jax.ShapeDtypeStruct((N, H), jnp.float32),
)


def _layer_body(h_ref, aggp_ref, degp_ref, b_ref, ws_ref, wn_ref,
                bc_ref, g_ref, be_ref, ge_ref, hn_ref, go_ref):
    i = pl.program_id(0)
    h = h_ref[...]
    agg = aggp_ref[0] + aggp_ref[1]
    deg = degp_ref[0, :, 0:1] + degp_ref[1, :, 0:1]
    r = 1.0 / jnp.maximum(deg, 1.0)
    nc = (jnp.dot(h, ws_ref[...], preferred_element_type=jnp.float32)
          + jnp.dot(agg * r, wn_ref[...], preferred_element_type=jnp.float32)
          + bc_ref[...])

    seg = lax.broadcasted_iota(jnp.int32, (BP, G), 1)
    oh = (b_ref[...] == seg).astype(jnp.float32)
    p = lax.dot_general(oh, nc, (((0,), (0,)), ((), ())),
                        preferred_element_type=jnp.float32)

    @pl.when(i == 0)
    def _():
        go_ref[...] = p

    @pl.when(i > 0)
    def _():
        go_ref[...] = go_ref[...] + p

    no = _ln(nc + h, g_ref[...], be_ref[...])
    hn_ref[...] = jnp.maximum(no, 0.0)

    @pl.when(i == pl.num_programs(0) - 1)
    def _():
        g0 = go_ref[...] + ge_ref[...]
        g0 = _ln(g0, g_ref[...], be_ref[...])
        go_ref[...] = _ln(g0, g_ref[...], be_ref[...])


_layer_call = pl.pallas_call(
    _layer_body,
    grid=(GRID,),
    in_specs=[
        pl.BlockSpec((BP, H), lambda i: (i, 0)),
        pl.BlockSpec((NC, BP, H), lambda i: (0, i, 0)),
        pl.BlockSpec((NC, BP, 16), lambda i: (0, i, 0)),
        pl.BlockSpec((BP, 1), lambda i: (i, 0)),
        pl.BlockSpec((H, H), lambda i: (0, 0)),
        pl.BlockSpec((H, H), lambda i: (0, 0)),
        pl.BlockSpec((1, H), lambda i: (0, 0)),
        pl.BlockSpec((1, H), lambda i: (0, 0)),
        pl.BlockSpec((1, H), lambda i: (0, 0)),
        pl.BlockSpec((G, H), lambda i: (0, 0)),
    ],
    out_specs=[
        pl.BlockSpec((BP, H), lambda i: (i, 0)),
        pl.BlockSpec((G, H), lambda i: (0, 0)),
    ],
    out_shape=[
        jax.ShapeDtypeStruct((N, H), jnp.float32),
        jax.ShapeDtypeStruct((G, H), jnp.float32),
    ],
)


def kernel(x, edge_index, batch, fc_W, fc_b, W_self, W_neigh, b_conv, gamma, beta):
    src = edge_index[0].astype(jnp.int32)
    dst = edge_index[1].astype(jnp.int32)
    pad = E_PAD - E
    pad_src = jnp.arange(pad, dtype=jnp.int32) % N
    src_p = jnp.concatenate([src, pad_src]).reshape(NW, CH, CHUNK)
    pad_dst = DUMMY + jnp.arange(pad, dtype=jnp.int32) % (N_ACC - N)
    dst_p = jnp.concatenate([dst, pad_dst]).reshape(NW, CH, CHUNK)
    batch_r = batch.astype(jnp.int32).reshape(N, 1)

    zeros16 = jnp.zeros((N_ACC, 16), jnp.float32)
    zeros64 = jnp.zeros((N_ACC, H), jnp.float32)
    ones16 = jnp.ones((CHUNK, 16), jnp.float32)

    deg_parts = _deg_call(dst_p, ones16, zeros16)
    h = _proj_call(x, fc_W, fc_b.reshape(1, H))
    ge = jnp.zeros((G, H), jnp.float32)
    for i in range(NB):
        agg_parts = _agg_call(h, src_p, dst_p, zeros64)
        h, ge = _layer_call(h, agg_parts, deg_parts, batch_r,
                            W_self[i], W_neigh[i],
                            b_conv[i].reshape(1, H),
                            gamma[i].reshape(1, H),
                            beta[i].reshape(1, H), ge)
    return (h, ge)


# speedup vs baseline: 1.2552x; 1.2552x over previous
import functools

import jax
import jax.numpy as jnp
from jax import lax
from jax.experimental import pallas as pl
from jax.experimental.pallas import tpu as pltpu
from jax.experimental.pallas import tpu_sc as plsc

N = 10000
E = 640000
D_IN = 128
H = 64
G = 64
NB = 3

NC = 2
NS = 16
NW = NC * NS

CHUNK = 128
CH = 160
CH2 = CH // 2
EPW = CH * CHUNK
E_PAD = NW * EPW
DUMMY = N
N_ACC = 10112
ZPT = N_ACC // NS
LAST = N - (NS - 1) * ZPT

BP = 2000
GRID = N // BP

_MESH = plsc.VectorSubcoreMesh(core_axis_name="c", subcore_axis_name="s")


def _deg_body(dst_hbm, ones_hbm, zero_hbm, out_hbm, dst_v, ones_v, acc_sh):
    c = lax.axis_index("c")
    s = lax.axis_index("s")
    w = c * NS + s
    pltpu.sync_copy(dst_hbm.at[w], dst_v)
    pltpu.sync_copy(ones_hbm, ones_v)
    pltpu.sync_copy(zero_hbm.at[pl.ds(s * ZPT, ZPT)], acc_sh.at[pl.ds(s * ZPT, ZPT)])
    plsc.subcore_barrier()

    def body(j, carry):
        pltpu.sync_copy(ones_v, acc_sh.at[dst_v.at[j]], add=True)
        return carry

    lax.fori_loop(0, CH2, body, 0)
    plsc.subcore_barrier()

    @pl.when(s < NS - 1)
    def _():
        pltpu.sync_copy(acc_sh.at[pl.ds(s * ZPT, ZPT)],
                        out_hbm.at[c, pl.ds(s * ZPT, ZPT)])

    @pl.when(s == NS - 1)
    def _():
        pltpu.sync_copy(acc_sh.at[pl.ds((NS - 1) * ZPT, LAST)],
                        out_hbm.at[c, pl.ds((NS - 1) * ZPT, LAST)])


_deg_call = pl.kernel(
    _deg_body,
    out_type=jax.ShapeDtypeStruct((NC, N, 16), jnp.float32),
    mesh=_MESH,
    compiler_params=pltpu.CompilerParams(use_tc_tiling_on_sc=False),
    scratch_types=[
        pltpu.VMEM((CH2, 2 * CHUNK), jnp.int32),
        pltpu.VMEM((2 * CHUNK, 16), jnp.float32),
        pltpu.VMEM_SHARED((N_ACC, 16), jnp.float32),
    ],
)


def _agg_body(h_hbm, src_hbm, dst_hbm, zero_hbm, out_hbm,
              src_v, dst_v, rows0, rows1, acc_sh, g0, g1):
    c = lax.axis_index("c")
    s = lax.axis_index("s")
    w = c * NS + s
    pltpu.sync_copy(src_hbm.at[w], src_v)
    pltpu.sync_copy(dst_hbm.at[w], dst_v)
    pltpu.sync_copy(zero_hbm.at[pl.ds(s * ZPT, ZPT)], acc_sh.at[pl.ds(s * ZPT, ZPT)])
    plsc.subcore_barrier()

    pltpu.async_copy(h_hbm.at[src_v.at[0]], rows0, g0)

    def body(u, carry):
        j0 = 2 * u
        j1 = 2 * u + 1
        pltpu.make_async_copy(h_hbm.at[src_v.at[j0]], rows0, g0).wait()
        pltpu.async_copy(h_hbm.at[src_v.at[j1]], rows1, g1)
        pltpu.sync_copy(rows0, acc_sh.at[dst_v.at[j0]], add=True)
        pltpu.make_async_copy(h_hbm.at[src_v.at[j1]], rows1, g1).wait()
        pltpu.async_copy(h_hbm.at[src_v.at[jnp.minimum(j1 + 1, CH2 - 1)]], rows0, g0)
        pltpu.sync_copy(rows1, acc_sh.at[dst_v.at[j1]], add=True)
        return carry

    lax.fori_loop(0, CH2 // 2, body, 0)
    pltpu.make_async_copy(h_hbm.at[src_v.at[CH2 - 1]], rows0, g0).wait()
    plsc.subcore_barrier()

    @pl.when(s < NS - 1)
    def _():
        pltpu.sync_copy(acc_sh.at[pl.ds(s * ZPT, ZPT)],
                        out_hbm.at[c, pl.ds(s * ZPT, ZPT)])

    @pl.when(s == NS - 1)
    def _():
        pltpu.sync_copy(acc_sh.at[pl.ds((NS - 1) * ZPT, LAST)],
                        out_hbm.at[c, pl.ds((NS - 1) * ZPT, LAST)])


_agg_call = pl.kernel(
    _agg_body,
    out_type=jax.ShapeDtypeStruct((NC, N, H), jnp.float32),
    mesh=_MESH,
    compiler_params=pltpu.CompilerParams(use_tc_tiling_on_sc=False),
    scratch_types=[
        pltpu.VMEM((CH2, 2 * CHUNK), jnp.int32),
        pltpu.VMEM((CH2, 2 * CHUNK), jnp.int32),
        pltpu.VMEM((2 * CHUNK, H), jnp.float32),
        pltpu.VMEM((2 * CHUNK, H), jnp.float32),
        pltpu.VMEM_SHARED((N_ACC, H), jnp.float32),
        pltpu.SemaphoreType.DMA,
        pltpu.SemaphoreType.DMA,
    ],
)


def _ln(x, g, b, eps=1e-5):
    m = jnp.mean(x, axis=-1, keepdims=True)
    v = jnp.mean((x - m) * (x - m), axis=-1, keepdims=True)
    return (x - m) / jnp.sqrt(v + eps) * g + b


def _proj_body(x_ref, w_ref, b_ref, h_ref):
    h_ref[...] = (jnp.dot(x_ref[...], w_ref[...],
                          preferred_element_type=jnp.float32) + b_ref[...])


_proj_call = pl.pallas_call(
    _proj_body,
    grid=(GRID,),
    in_specs=[
        pl.BlockSpec((BP, D_IN), lambda i: (i, 0)),
        pl.BlockSpec((D_IN, H), lambda i: (0, 0)),
        pl.BlockSpec((1, H), lambda i: (0, 0)),
    ],
    out_specs=pl.BlockSpec((BP, H), lambda i: (i, 0)),
    out_shape=jax.ShapeDtypeStruct((N, H), jnp.float32),
)


def _layer_body(h_ref, aggp_ref, degp_ref, b_ref, ws_ref, wn_ref,
                bc_ref, g_ref, be_ref, ge_ref, hn_ref, go_ref):
    i = pl.program_id(0)
    h = h_ref[...]
    agg = aggp_ref[0] + aggp_ref[1]
    deg = degp_ref[0, :, 0:1] + degp_ref[1, :, 0:1]
    r = 1.0 / jnp.maximum(deg, 1.0)
    nc = (jnp.dot(h, ws_ref[...], preferred_element_type=jnp.float32)
          + jnp.dot(agg * r, wn_ref[...], preferred_element_type=jnp.float32)
          + bc_ref[...])

    seg = lax.broadcasted_iota(jnp.int32, (BP, G), 1)
    oh = (b_ref[...] == seg).astype(jnp.float32)
    p = lax.dot_general(oh, nc, (((0,), (0,)), ((), ())),
                        preferred_element_type=jnp.float32)

    @pl.when(i == 0)
    def _():
        go_ref[...] = p

    @pl.when(i > 0)
    def _():
        go_ref[...] = go_ref[...] + p

    no = _ln(nc + h, g_ref[...], be_ref[...])
    hn_ref[...] = jnp.maximum(no, 0.0)

    @pl.when(i == pl.num_programs(0) - 1)
    def _():
        g0 = go_ref[...] + ge_ref[...]
        g0 = _ln(g0, g_ref[...], be_ref[...])
        go_ref[...] = _ln(g0, g_ref[...], be_ref[...])


_layer_call = pl.pallas_call(
    _layer_body,
    grid=(GRID,),
    in_specs=[
        pl.BlockSpec((BP, H), lambda i: (i, 0)),
        pl.BlockSpec((NC, BP, H), lambda i: (0, i, 0)),
        pl.BlockSpec((NC, BP, 16), lambda i: (0, i, 0)),
        pl.BlockSpec((BP, 1), lambda i: (i, 0)),
        pl.BlockSpec((H, H), lambda i: (0, 0)),
        pl.BlockSpec((H, H), lambda i: (0, 0)),
        pl.BlockSpec((1, H), lambda i: (0, 0)),
        pl.BlockSpec((1, H), lambda i: (0, 0)),
        pl.BlockSpec((1, H), lambda i: (0, 0)),
        pl.BlockSpec((G, H), lambda i: (0, 0)),
    ],
    out_specs=[
        pl.BlockSpec((BP, H), lambda i: (i, 0)),
        pl.BlockSpec((G, H), lambda i: (0, 0)),
    ],
    out_shape=[
        jax.ShapeDtypeStruct((N, H), jnp.float32),
        jax.ShapeDtypeStruct((G, H), jnp.float32),
    ],
)


def kernel(x, edge_index, batch, fc_W, fc_b, W_self, W_neigh, b_conv, gamma, beta):
    src = edge_index[0].astype(jnp.int32)
    dst = edge_index[1].astype(jnp.int32)
    pad = E_PAD - E
    pad_src = jnp.arange(pad, dtype=jnp.int32) % N
    src_p = jnp.concatenate([src, pad_src]).reshape(NW, CH2, 2 * CHUNK)
    pad_dst = DUMMY + jnp.arange(pad, dtype=jnp.int32) % (N_ACC - N)
    dst_p = jnp.concatenate([dst, pad_dst]).reshape(NW, CH2, 2 * CHUNK)
    batch_r = batch.astype(jnp.int32).reshape(N, 1)

    zeros16 = jnp.zeros((N_ACC, 16), jnp.float32)
    zeros64 = jnp.zeros((N_ACC, H), jnp.float32)
    ones16 = jnp.ones((2 * CHUNK, 16), jnp.float32)

    deg_parts = _deg_call(dst_p, ones16, zeros16)
    h = _proj_call(x, fc_W, fc_b.reshape(1, H))
    ge = jnp.zeros((G, H), jnp.float32)
    for i in range(NB):
        agg_parts = _agg_call(h, src_p, dst_p, zeros64)
        h, ge = _layer_call(h, agg_parts, deg_parts, batch_r,
                            W_self[i], W_neigh[i],
                            b_conv[i].reshape(1, H),
                            gamma[i].reshape(1, H),
                            beta[i].reshape(1, H), ge)
    return (h, ge)


# speedup vs baseline: 1.3561x; 1.0803x over previous
import functools

import jax
import jax.numpy as jnp
from jax import lax
from jax.experimental import pallas as pl
from jax.experimental.pallas import tpu as pltpu
from jax.experimental.pallas import tpu_sc as plsc

N = 10000
E = 640000
D_IN = 128
H = 64
G = 64
NB = 3

NC = 2
NS = 16
NW = NC * NS

CHUNK = 128
CH = 160
ROWS = 512
EPW = CH * CHUNK
E_PAD = NW * EPW
NJ = EPW // ROWS
DUMMY = N
N_ACC = 10112
ZPT = N_ACC // NS
LAST = N - (NS - 1) * ZPT

BP = 2000
GRID = N // BP

_MESH = plsc.VectorSubcoreMesh(core_axis_name="c", subcore_axis_name="s")


def _deg_body(dst_hbm, ones_hbm, zero_hbm, out_hbm, dst_v, ones_v, acc_sh):
    c = lax.axis_index("c")
    s = lax.axis_index("s")
    w = c * NS + s
    pltpu.sync_copy(dst_hbm.at[w], dst_v)
    pltpu.sync_copy(ones_hbm, ones_v)
    pltpu.sync_copy(zero_hbm.at[pl.ds(s * ZPT, ZPT)], acc_sh.at[pl.ds(s * ZPT, ZPT)])
    plsc.subcore_barrier()

    def body(j, carry):
        pltpu.sync_copy(ones_v, acc_sh.at[dst_v.at[j]], add=True)
        return carry

    lax.fori_loop(0, NJ, body, 0)
    plsc.subcore_barrier()

    @pl.when(s < NS - 1)
    def _():
        pltpu.sync_copy(acc_sh.at[pl.ds(s * ZPT, ZPT)],
                        out_hbm.at[c, pl.ds(s * ZPT, ZPT)])

    @pl.when(s == NS - 1)
    def _():
        pltpu.sync_copy(acc_sh.at[pl.ds((NS - 1) * ZPT, LAST)],
                        out_hbm.at[c, pl.ds((NS - 1) * ZPT, LAST)])


_deg_call = pl.kernel(
    _deg_body,
    out_type=jax.ShapeDtypeStruct((NC, N, 16), jnp.float32),
    mesh=_MESH,
    compiler_params=pltpu.CompilerParams(use_tc_tiling_on_sc=False),
    scratch_types=[
        pltpu.VMEM((NJ, ROWS), jnp.int32),
        pltpu.VMEM((ROWS, 16), jnp.float32),
        pltpu.VMEM_SHARED((N_ACC, 16), jnp.float32),
    ],
)


def _agg_body(h_hbm, src_hbm, dst_hbm, zero_hbm, out_hbm,
              si0, si1, di0, di1, rows0, rows1, acc_sh, g0, g1, i0, i1):
    c = lax.axis_index("c")
    s = lax.axis_index("s")
    w = c * NS + s
    pltpu.sync_copy(zero_hbm.at[pl.ds(s * ZPT, ZPT)], acc_sh.at[pl.ds(s * ZPT, ZPT)])
    pltpu.sync_copy(src_hbm.at[w, 0], si0)
    pltpu.sync_copy(dst_hbm.at[w, 0], di0)
    plsc.subcore_barrier()

    pltpu.async_copy(h_hbm.at[si0], rows0, g0)
    pltpu.async_copy(src_hbm.at[w, 1], si1, i0)
    pltpu.async_copy(dst_hbm.at[w, 1], di1, i0)

    def body(u, carry):
        j0 = 2 * u
        j1 = 2 * u + 1
        pltpu.make_async_copy(src_hbm.at[w, j1], si1, i0).wait()
        pltpu.make_async_copy(dst_hbm.at[w, j1], di1, i0).wait()
        pltpu.make_async_copy(h_hbm.at[si0], rows0, g0).wait()
        pltpu.async_copy(h_hbm.at[si1], rows1, g1)
        pltpu.sync_copy(rows0, acc_sh.at[di0], add=True)
        nxt = jnp.minimum(j0 + 2, NJ - 1)
        pltpu.async_copy(src_hbm.at[w, nxt], si0, i1)
        pltpu.async_copy(dst_hbm.at[w, nxt], di0, i1)
        pltpu.make_async_copy(h_hbm.at[si1], rows1, g1).wait()
        pltpu.make_async_copy(src_hbm.at[w, nxt], si0, i1).wait()
        pltpu.make_async_copy(dst_hbm.at[w, nxt], di0, i1).wait()
        pltpu.async_copy(h_hbm.at[si0], rows0, g0)
        pltpu.sync_copy(rows1, acc_sh.at[di1], add=True)
        nx2 = jnp.minimum(j1 + 2, NJ - 1)
        pltpu.async_copy(src_hbm.at[w, nx2], si1, i0)
        pltpu.async_copy(dst_hbm.at[w, nx2], di1, i0)
        return carry

    lax.fori_loop(0, NJ // 2, body, 0)
    pltpu.make_async_copy(src_hbm.at[w, NJ - 1], si1, i0).wait()
    pltpu.make_async_copy(dst_hbm.at[w, NJ - 1], di1, i0).wait()
    pltpu.make_async_copy(h_hbm.at[si0], rows0, g0).wait()
    plsc.subcore_barrier()

    @pl.when(s < NS - 1)
    def _():
        pltpu.sync_copy(acc_sh.at[pl.ds(s * ZPT, ZPT)],
                        out_hbm.at[c, pl.ds(s * ZPT, ZPT)])

    @pl.when(s == NS - 1)
    def _():
        pltpu.sync_copy(acc_sh.at[pl.ds((NS - 1) * ZPT, LAST)],
                        out_hbm.at[c, pl.ds((NS - 1) * ZPT, LAST)])


_agg_call = pl.kernel(
    _agg_body,
    out_type=jax.ShapeDtypeStruct((NC, N, H), jnp.float32),
    mesh=_MESH,
    compiler_params=pltpu.CompilerParams(use_tc_tiling_on_sc=False),
    scratch_types=[
        pltpu.VMEM((ROWS,), jnp.int32),
        pltpu.VMEM((ROWS,), jnp.int32),
        pltpu.VMEM((ROWS,), jnp.int32),
        pltpu.VMEM((ROWS,), jnp.int32),
        pltpu.VMEM((ROWS, H), jnp.float32),
        pltpu.VMEM((ROWS, H), jnp.float32),
        pltpu.VMEM_SHARED((N_ACC, H), jnp.float32),
        pltpu.SemaphoreType.DMA,
        pltpu.SemaphoreType.DMA,
        pltpu.SemaphoreType.DMA,
        pltpu.SemaphoreType.DMA,
    ],
)


def _ln(x, g, b, eps=1e-5):
    m = jnp.mean(x, axis=-1, keepdims=True)
    v = jnp.mean((x - m) * (x - m), axis=-1, keepdims=True)
    return (x - m) / jnp.sqrt(v + eps) * g + b


def _proj_body(x_ref, w_ref, b_ref, h_ref):
    h_ref[...] = (jnp.dot(x_ref[...], w_ref[...],
                          preferred_element_type=jnp.float32) + b_ref[...])


_proj_call = pl.pallas_call(
    _proj_body,
    grid=(GRID,),
    in_specs=[
        pl.BlockSpec((BP, D_IN), lambda i: (i, 0)),
        pl.BlockSpec((D_IN, H), lambda i: (0, 0)),
        pl.BlockSpec((1, H), lambda i: (0, 0)),
    ],
    out_specs=pl.BlockSpec((BP, H), lambda i: (i, 0)),
    out_shape=jax.ShapeDtypeStruct((N, H), jnp.float32),
)


def _layer_body(h_ref, aggp_ref, degp_ref, b_ref, ws_ref, wn_ref,
                bc_ref, g_ref, be_ref, ge_ref, hn_ref, go_ref):
    i = pl.program_id(0)
    h = h_ref[...]
    agg = aggp_ref[0] + aggp_ref[1]
    deg = degp_ref[0, :, 0:1] + degp_ref[1, :, 0:1]
    r = 1.0 / jnp.maximum(deg, 1.0)
    nc = (jnp.dot(h, ws_ref[...], preferred_element_type=jnp.float32)
          + jnp.dot(agg * r, wn_ref[...], preferred_element_type=jnp.float32)
          + bc_ref[...])

    seg = lax.broadcasted_iota(jnp.int32, (BP, G), 1)
    oh = (b_ref[...] == seg).astype(jnp.float32)
    p = lax.dot_general(oh, nc, (((0,), (0,)), ((), ())),
                        preferred_element_type=jnp.float32)

    @pl.when(i == 0)
    def _():
        go_ref[...] = p

    @pl.when(i > 0)
    def _():
        go_ref[...] = go_ref[...] + p

    no = _ln(nc + h, g_ref[...], be_ref[...])
    hn_ref[...] = jnp.maximum(no, 0.0)

    @pl.when(i == pl.num_programs(0) - 1)
    def _():
        g0 = go_ref[...] + ge_ref[...]
        g0 = _ln(g0, g_ref[...], be_ref[...])
        go_ref[...] = _ln(g0, g_ref[...], be_ref[...])


_layer_call = pl.pallas_call(
    _layer_body,
    grid=(GRID,),
    in_specs=[
        pl.BlockSpec((BP, H), lambda i: (i, 0)),
        pl.BlockSpec((NC, BP, H), lambda i: (0, i, 0)),
        pl.BlockSpec((NC, BP, 16), lambda i: (0, i, 0)),
        pl.BlockSpec((BP, 1), lambda i: (i, 0)),
        pl.BlockSpec((H, H), lambda i: (0, 0)),
        pl.BlockSpec((H, H), lambda i: (0, 0)),
        pl.BlockSpec((1, H), lambda i: (0, 0)),
        pl.BlockSpec((1, H), lambda i: (0, 0)),
        pl.BlockSpec((1, H), lambda i: (0, 0)),
        pl.BlockSpec((G, H), lambda i: (0, 0)),
    ],
    out_specs=[
        pl.BlockSpec((BP, H), lambda i: (i, 0)),
        pl.BlockSpec((G, H), lambda i: (0, 0)),
    ],
    out_shape=[
        jax.ShapeDtypeStruct((N, H), jnp.float32),
        jax.ShapeDtypeStruct((G, H), jnp.float32),
    ],
)


def kernel(x, edge_index, batch, fc_W, fc_b, W_self, W_neigh, b_conv, gamma, beta):
    src = edge_index[0].astype(jnp.int32)
    dst = edge_index[1].astype(jnp.int32)
    pad = E_PAD - E
    pad_src = jnp.arange(pad, dtype=jnp.int32) % N
    src_p = jnp.concatenate([src, pad_src]).reshape(NW, NJ, ROWS)
    pad_dst = DUMMY + jnp.arange(pad, dtype=jnp.int32) % (N_ACC - N)
    dst_p = jnp.concatenate([dst, pad_dst]).reshape(NW, NJ, ROWS)
    batch_r = batch.astype(jnp.int32).reshape(N, 1)

    zeros16 = jnp.zeros((N_ACC, 16), jnp.float32)
    zeros64 = jnp.zeros((N_ACC, H), jnp.float32)
    ones16 = jnp.ones((ROWS, 16), jnp.float32)

    deg_parts = _deg_call(dst_p, ones16, zeros16)
    h = _proj_call(x, fc_W, fc_b.reshape(1, H))
    ge = jnp.zeros((G, H), jnp.float32)
    for i in range(NB):
        agg_parts = _agg_call(h, src_p, dst_p, zeros64)
        h, ge = _layer_call(h, agg_parts, deg_parts, batch_r,
                            W_self[i], W_neigh[i],
                            b_conv[i].reshape(1, H),
                            gamma[i].reshape(1, H),
                            beta[i].reshape(1, H), ge)
    return (h, ge)


# speedup vs baseline: 1.3713x; 1.0112x over previous
import functools

import jax
import jax.numpy as jnp
from jax import lax
from jax.experimental import pallas as pl
from jax.experimental.pallas import tpu as pltpu
from jax.experimental.pallas import tpu_sc as plsc

N = 10000
E = 640000
D_IN = 128
H = 64
G = 64
NB = 3

NC = 2
NS = 16
NW = NC * NS

CHUNK = 128
CH = 160
ROWS = 640
EPW = CH * CHUNK
E_PAD = NW * EPW
NJ = EPW // ROWS
DUMMY = N
N_ACC = 10112
ZPT = N_ACC // NS
LAST = N - (NS - 1) * ZPT

BP = 2000
GRID = N // BP

_MESH = plsc.VectorSubcoreMesh(core_axis_name="c", subcore_axis_name="s")


def _deg_body(dst_hbm, ones_hbm, zero_hbm, out_hbm, dst_v, ones_v, acc_sh):
    c = lax.axis_index("c")
    s = lax.axis_index("s")
    w = c * NS + s
    pltpu.sync_copy(dst_hbm.at[w], dst_v)
    pltpu.sync_copy(ones_hbm, ones_v)
    pltpu.sync_copy(zero_hbm.at[pl.ds(s * ZPT, ZPT)], acc_sh.at[pl.ds(s * ZPT, ZPT)])
    plsc.subcore_barrier()

    def body(j, carry):
        pltpu.sync_copy(ones_v, acc_sh.at[dst_v.at[j]], add=True)
        return carry

    lax.fori_loop(0, NJ, body, 0)
    plsc.subcore_barrier()

    @pl.when(s < NS - 1)
    def _():
        pltpu.sync_copy(acc_sh.at[pl.ds(s * ZPT, ZPT)],
                        out_hbm.at[c, pl.ds(s * ZPT, ZPT)])

    @pl.when(s == NS - 1)
    def _():
        pltpu.sync_copy(acc_sh.at[pl.ds((NS - 1) * ZPT, LAST)],
                        out_hbm.at[c, pl.ds((NS - 1) * ZPT, LAST)])


_deg_call = pl.kernel(
    _deg_body,
    out_type=jax.ShapeDtypeStruct((NC, N, 16), jnp.float32),
    mesh=_MESH,
    compiler_params=pltpu.CompilerParams(use_tc_tiling_on_sc=False),
    scratch_types=[
        pltpu.VMEM((NJ, ROWS), jnp.int32),
        pltpu.VMEM((ROWS, 16), jnp.float32),
        pltpu.VMEM_SHARED((N_ACC, 16), jnp.float32),
    ],
)


def _agg_body(h_hbm, src_hbm, dst_hbm, zero_hbm, out_hbm,
              si0, si1, di0, di1, rows0, rows1, acc_sh, g0, g1, i0, i1):
    c = lax.axis_index("c")
    s = lax.axis_index("s")
    w = c * NS + s
    pltpu.sync_copy(zero_hbm.at[pl.ds(s * ZPT, ZPT)], acc_sh.at[pl.ds(s * ZPT, ZPT)])
    pltpu.sync_copy(src_hbm.at[w, 0], si0)
    pltpu.sync_copy(dst_hbm.at[w, 0], di0)
    plsc.subcore_barrier()

    pltpu.async_copy(h_hbm.at[si0], rows0, g0)
    pltpu.async_copy(src_hbm.at[w, 1], si1, i0)
    pltpu.async_copy(dst_hbm.at[w, 1], di1, i0)

    def body(u, carry):
        j0 = 2 * u
        j1 = 2 * u + 1
        pltpu.make_async_copy(src_hbm.at[w, j1], si1, i0).wait()
        pltpu.make_async_copy(dst_hbm.at[w, j1], di1, i0).wait()
        pltpu.make_async_copy(h_hbm.at[si0], rows0, g0).wait()
        pltpu.async_copy(h_hbm.at[si1], rows1, g1)
        pltpu.sync_copy(rows0, acc_sh.at[di0], add=True)
        nxt = jnp.minimum(j0 + 2, NJ - 1)
        pltpu.async_copy(src_hbm.at[w, nxt], si0, i1)
        pltpu.async_copy(dst_hbm.at[w, nxt], di0, i1)
        pltpu.make_async_copy(h_hbm.at[si1], rows1, g1).wait()
        pltpu.make_async_copy(src_hbm.at[w, nxt], si0, i1).wait()
        pltpu.make_async_copy(dst_hbm.at[w, nxt], di0, i1).wait()
        pltpu.async_copy(h_hbm.at[si0], rows0, g0)
        pltpu.sync_copy(rows1, acc_sh.at[di1], add=True)
        nx2 = jnp.minimum(j1 + 2, NJ - 1)
        pltpu.async_copy(src_hbm.at[w, nx2], si1, i0)
        pltpu.async_copy(dst_hbm.at[w, nx2], di1, i0)
        return carry

    lax.fori_loop(0, NJ // 2, body, 0)
    pltpu.make_async_copy(src_hbm.at[w, NJ - 1], si1, i0).wait()
    pltpu.make_async_copy(dst_hbm.at[w, NJ - 1], di1, i0).wait()
    pltpu.make_async_copy(h_hbm.at[si0], rows0, g0).wait()
    plsc.subcore_barrier()

    @pl.when(s < NS - 1)
    def _():
        pltpu.sync_copy(acc_sh.at[pl.ds(s * ZPT, ZPT)],
                        out_hbm.at[c, pl.ds(s * ZPT, ZPT)])

    @pl.when(s == NS - 1)
    def _():
        pltpu.sync_copy(acc_sh.at[pl.ds((NS - 1) * ZPT, LAST)],
                        out_hbm.at[c, pl.ds((NS - 1) * ZPT, LAST)])


_agg_call = pl.kernel(
    _agg_body,
    out_type=jax.ShapeDtypeStruct((NC, N, H), jnp.float32),
    mesh=_MESH,
    compiler_params=pltpu.CompilerParams(use_tc_tiling_on_sc=False),
    scratch_types=[
        pltpu.VMEM((ROWS,), jnp.int32),
        pltpu.VMEM((ROWS,), jnp.int32),
        pltpu.VMEM((ROWS,), jnp.int32),
        pltpu.VMEM((ROWS,), jnp.int32),
        pltpu.VMEM((ROWS, H), jnp.float32),
        pltpu.VMEM((ROWS, H), jnp.float32),
        pltpu.VMEM_SHARED((N_ACC, H), jnp.float32),
        pltpu.SemaphoreType.DMA,
        pltpu.SemaphoreType.DMA,
        pltpu.SemaphoreType.DMA,
        pltpu.SemaphoreType.DMA,
    ],
)


def _ln(x, g, b, eps=1e-5):
    m = jnp.mean(x, axis=-1, keepdims=True)
    v = jnp.mean((x - m) * (x - m), axis=-1, keepdims=True)
    return (x - m) / jnp.sqrt(v + eps) * g + b


def _proj_body(x_ref, w_ref, b_ref, h_ref):
    h_ref[...] = (jnp.dot(x_ref[...], w_ref[...],
                          preferred_element_type=jnp.float32) + b_ref[...])


_proj_call = pl.pallas_call(
    _proj_body,
    grid=(GRID,),
    in_specs=[
        pl.BlockSpec((BP, D_IN), lambda i: (i, 0)),
        pl.BlockSpec((D_IN, H), lambda i: (0, 0)),
        pl.BlockSpec((1, H), lambda i: (0, 0)),
    ],
    out_specs=pl.BlockSpec((BP, H), lambda i: (i, 0)),
    out_shape=jax.ShapeDtypeStruct((N, H), jnp.float32),
)


def _layer_body(h_ref, aggp_ref, degp_ref, b_ref, ws_ref, wn_ref,
                bc_ref, g_ref, be_ref, ge_ref, hn_ref, go_ref):
    i = pl.program_id(0)
    h = h_ref[...]
    agg = aggp_ref[0] + aggp_ref[1]
    deg = degp_ref[0, :, 0:1] + degp_ref[1, :, 0:1]
    r = 1.0 / jnp.maximum(deg, 1.0)
    nc = (jnp.dot(h, ws_ref[...], preferred_element_type=jnp.float32)
          + jnp.dot(agg * r, wn_ref[...], preferred_element_type=jnp.float32)
          + bc_ref[...])

    seg = lax.broadcasted_iota(jnp.int32, (BP, G), 1)
    oh = (b_ref[...] == seg).astype(jnp.float32)
    p = lax.dot_general(oh, nc, (((0,), (0,)), ((), ())),
                        preferred_element_type=jnp.float32)

    @pl.when(i == 0)
    def _():
        go_ref[...] = p

    @pl.when(i > 0)
    def _():
        go_ref[...] = go_ref[...] + p

    no = _ln(nc + h, g_ref[...], be_ref[...])
    hn_ref[...] = jnp.maximum(no, 0.0)

    @pl.when(i == pl.num_programs(0) - 1)
    def _():
        g0 = go_ref[...] + ge_ref[...]
        g0 = _ln(g0, g_ref[...], be_ref[...])
        go_ref[...] = _ln(g0, g_ref[...], be_ref[...])


_layer_call = pl.pallas_call(
    _layer_body,
    grid=(GRID,),
    in_specs=[
        pl.BlockSpec((BP, H), lambda i: (i, 0)),
        pl.BlockSpec((NC, BP, H), lambda i: (0, i, 0)),
        pl.BlockSpec((NC, BP, 16), lambda i: (0, i, 0)),
        pl.BlockSpec((BP, 1), lambda i: (i, 0)),
        pl.BlockSpec((H, H), lambda i: (0, 0)),
        pl.BlockSpec((H, H), lambda i: (0, 0)),
        pl.BlockSpec((1, H), lambda i: (0, 0)),
        pl.BlockSpec((1, H), lambda i: (0, 0)),
        pl.BlockSpec((1, H), lambda i: (0, 0)),
        pl.BlockSpec((G, H), lambda i: (0, 0)),
    ],
    out_specs=[
        pl.BlockSpec((BP, H), lambda i: (i, 0)),
        pl.BlockSpec((G, H), lambda i: (0, 0)),
    ],
    out_shape=[
        jax.ShapeDtypeStruct((N, H), jnp.float32),
        jax.ShapeDtypeStruct((G, H), jnp.float32),
    ],
)


def kernel(x, edge_index, batch, fc_W, fc_b, W_self, W_neigh, b_conv, gamma, beta):
    src = edge_index[0].astype(jnp.int32)
    dst = edge_index[1].astype(jnp.int32)
    pad = E_PAD - E
    pad_src = jnp.arange(pad, dtype=jnp.int32) % N
    src_p = jnp.concatenate([src, pad_src]).reshape(NW, NJ, ROWS)
    pad_dst = DUMMY + jnp.arange(pad, dtype=jnp.int32) % (N_ACC - N)
    dst_p = jnp.concatenate([dst, pad_dst]).reshape(NW, NJ, ROWS)
    batch_r = batch.astype(jnp.int32).reshape(N, 1)

    zeros16 = jnp.zeros((N_ACC, 16), jnp.float32)
    zeros64 = jnp.zeros((N_ACC, H), jnp.float32)
    ones16 = jnp.ones((ROWS, 16), jnp.float32)

    deg_parts = _deg_call(dst_p, ones16, zeros16)
    h = _proj_call(x, fc_W, fc_b.reshape(1, H))
    ge = jnp.zeros((G, H), jnp.float32)
    for i in range(NB):
        agg_parts = _agg_call(h, src_p, dst_p, zeros64)
        h, ge = _layer_call(h, agg_parts, deg_parts, batch_r,
                            W_self[i], W_neigh[i],
                            b_conv[i].reshape(1, H),
                            gamma[i].reshape(1, H),
                            beta[i].reshape(1, H), ge)
    return (h, ge)


# speedup vs baseline: 1.3722x; 1.0007x over previous
import jax
import jax.numpy as jnp
from jax import lax
from jax.experimental import pallas as pl
from jax.experimental.pallas import tpu as pltpu
from jax.experimental.pallas import tpu_sc as plsc

N = 10000
E = 640000
D_IN = 128
H = 64
G = 64
NB = 3

NC = 2
NS = 16
NW = NC * NS

CHUNK = 128
CH = 160
ROWS = 640
EPW = CH * CHUNK
E_PAD = NW * EPW
NJ = EPW // ROWS
DUMMY = N
N_ACC = 10112
ZPT = N_ACC // NS
LAST = N - (NS - 1) * ZPT

BP = 2000
GRID = N // BP

_MESH = plsc.VectorSubcoreMesh(core_axis_name="c", subcore_axis_name="s")


def _deg_body(dst_hbm, ones_hbm, zero_hbm, out_hbm, dst_v, ones_v, acc_sh):
    c = lax.axis_index("c")
    s = lax.axis_index("s")
    w = c * NS + s
    pltpu.sync_copy(dst_hbm.at[w], dst_v)
    pltpu.sync_copy(ones_hbm, ones_v)
    pltpu.sync_copy(zero_hbm.at[pl.ds(s * ZPT, ZPT)], acc_sh.at[pl.ds(s * ZPT, ZPT)])
    plsc.subcore_barrier()

    def body(j, carry):
        pltpu.sync_copy(ones_v, acc_sh.at[dst_v.at[j]], add=True)
        return carry

    lax.fori_loop(0, NJ, body, 0)
    plsc.subcore_barrier()

    @pl.when(s < NS - 1)
    def _():
        pltpu.sync_copy(acc_sh.at[pl.ds(s * ZPT, ZPT)],
                        out_hbm.at[c, pl.ds(s * ZPT, ZPT)])

    @pl.when(s == NS - 1)
    def _():
        pltpu.sync_copy(acc_sh.at[pl.ds((NS - 1) * ZPT, LAST)],
                        out_hbm.at[c, pl.ds((NS - 1) * ZPT, LAST)])


_deg_call = pl.kernel(
    _deg_body,
    out_type=jax.ShapeDtypeStruct((NC, N, 16), jnp.float32),
    mesh=_MESH,
    compiler_params=pltpu.CompilerParams(use_tc_tiling_on_sc=False),
    scratch_types=[
        pltpu.VMEM((NJ, ROWS), jnp.int32),
        pltpu.VMEM((ROWS, 16), jnp.float32),
        pltpu.VMEM_SHARED((N_ACC, 16), jnp.float32),
    ],
)


def _agg_body(h_hbm, src_hbm, dst_hbm, zero_hbm, out_hbm,
              si0, si1, di0, di1, rows0, rows1, acc_sh, g0, g1, i0, i1):
    c = lax.axis_index("c")
    s = lax.axis_index("s")
    w = c * NS + s
    pltpu.sync_copy(zero_hbm.at[pl.ds(s * ZPT, ZPT)], acc_sh.at[pl.ds(s * ZPT, ZPT)])
    pltpu.sync_copy(src_hbm.at[w, 0], si0)
    pltpu.sync_copy(dst_hbm.at[w, 0], di0)
    plsc.subcore_barrier()

    pltpu.async_copy(h_hbm.at[si0], rows0, g0)
    pltpu.async_copy(src_hbm.at[w, 1], si1, i0)
    pltpu.async_copy(dst_hbm.at[w, 1], di1, i0)

    def body(u, carry):
        j0 = 2 * u
        j1 = 2 * u + 1
        pltpu.make_async_copy(src_hbm.at[w, j1], si1, i0).wait()
        pltpu.make_async_copy(dst_hbm.at[w, j1], di1, i0).wait()
        pltpu.make_async_copy(h_hbm.at[si0], rows0, g0).wait()
        pltpu.async_copy(h_hbm.at[si1], rows1, g1)
        pltpu.sync_copy(rows0, acc_sh.at[di0], add=True)
        nxt = jnp.minimum(j0 + 2, NJ - 1)
        pltpu.async_copy(src_hbm.at[w, nxt], si0, i1)
        pltpu.async_copy(dst_hbm.at[w, nxt], di0, i1)
        pltpu.make_async_copy(h_hbm.at[si1], rows1, g1).wait()
        pltpu.make_async_copy(src_hbm.at[w, nxt], si0, i1).wait()
        pltpu.make_async_copy(dst_hbm.at[w, nxt], di0, i1).wait()
        pltpu.async_copy(h_hbm.at[si0], rows0, g0)
        pltpu.sync_copy(rows1, acc_sh.at[di1], add=True)
        nx2 = jnp.minimum(j1 + 2, NJ - 1)
        pltpu.async_copy(src_hbm.at[w, nx2], si1, i0)
        pltpu.async_copy(dst_hbm.at[w, nx2], di1, i0)
        return carry

    lax.fori_loop(0, NJ // 2, body, 0)
    pltpu.make_async_copy(src_hbm.at[w, NJ - 1], si1, i0).wait()
    pltpu.make_async_copy(dst_hbm.at[w, NJ - 1], di1, i0).wait()
    pltpu.make_async_copy(h_hbm.at[si0], rows0, g0).wait()
    plsc.subcore_barrier()

    @pl.when(s < NS - 1)
    def _():
        pltpu.sync_copy(acc_sh.at[pl.ds(s * ZPT, ZPT)],
                        out_hbm.at[c, pl.ds(s * ZPT, ZPT)])

    @pl.when(s == NS - 1)
    def _():
        pltpu.sync_copy(acc_sh.at[pl.ds((NS - 1) * ZPT, LAST)],
                        out_hbm.at[c, pl.ds((NS - 1) * ZPT, LAST)])


_agg_call = pl.kernel(
    _agg_body,
    out_type=jax.ShapeDtypeStruct((NC, N, H), jnp.float32),
    mesh=_MESH,
    compiler_params=pltpu.CompilerParams(use_tc_tiling_on_sc=False),
    scratch_types=[
        pltpu.VMEM((ROWS,), jnp.int32),
        pltpu.VMEM((ROWS,), jnp.int32),
        pltpu.VMEM((ROWS,), jnp.int32),
        pltpu.VMEM((ROWS,), jnp.int32),
        pltpu.VMEM((ROWS, H), jnp.float32),
        pltpu.VMEM((ROWS, H), jnp.float32),
        pltpu.VMEM_SHARED((N_ACC, H), jnp.float32),
        pltpu.SemaphoreType.DMA,
        pltpu.SemaphoreType.DMA,
        pltpu.SemaphoreType.DMA,
        pltpu.SemaphoreType.DMA,
    ],
)


def _ln(x, g, b, eps=1e-5):
    m = jnp.mean(x, axis=-1, keepdims=True)
    v = jnp.mean((x - m) * (x - m), axis=-1, keepdims=True)
    return (x - m) / jnp.sqrt(v + eps) * g + b


def _proj_body(x_ref, w_ref, b_ref, h_ref):
    h_ref[...] = (jnp.dot(x_ref[...], w_ref[...],
                          preferred_element_type=jnp.float32) + b_ref[...])


_proj_call = pl.pallas_call(
    _proj_body,
    grid=(GRID,),
    in_specs=[
        pl.BlockSpec((BP, D_IN), lambda i: (i, 0)),
        pl.BlockSpec((D_IN, H), lambda i: (0, 0)),
        pl.BlockSpec((1, H), lambda i: (0, 0)),
    ],
    out_specs=pl.BlockSpec((BP, H), lambda i: (i, 0)),
    out_shape=jax.ShapeDtypeStruct((N, H), jnp.float32),
)


def _layer_body(h_ref, aggp_ref, degp_ref, b_ref, ws_ref, wn_ref,
                bc_ref, g_ref, be_ref, ge_ref, hn_ref, go_ref):
    i = pl.program_id(0)
    h = h_ref[...]
    agg = aggp_ref[0] + aggp_ref[1]
    deg = degp_ref[0, :, 0:1] + degp_ref[1, :, 0:1]
    r = 1.0 / jnp.maximum(deg, 1.0)
    nc = (jnp.dot(h, ws_ref[...], preferred_element_type=jnp.float32)
          + jnp.dot(agg * r, wn_ref[...], preferred_element_type=jnp.float32)
          + bc_ref[...])

    seg = lax.broadcasted_iota(jnp.int32, (BP, G), 1)
    oh = (b_ref[...] == seg).astype(jnp.float32)
    p = lax.dot_general(oh, nc, (((0,), (0,)), ((), ())),
                        preferred_element_type=jnp.float32)

    @pl.when(i == 0)
    def _():
        go_ref[...] = p

    @pl.when(i > 0)
    def _():
        go_ref[...] = go_ref[...] + p

    no = _ln(nc + h, g_ref[...], be_ref[...])
    hn_ref[...] = jnp.maximum(no, 0.0)

    @pl.when(i == pl.num_programs(0) - 1)
    def _():
        g0 = go_ref[...] + ge_ref[...]
        g0 = _ln(g0, g_ref[...], be_ref[...])
        go_ref[...] = _ln(g0, g_ref[...], be_ref[...])


_layer_call = pl.pallas_call(
    _layer_body,
    grid=(GRID,),
    in_specs=[
        pl.BlockSpec((BP, H), lambda i: (i, 0)),
        pl.BlockSpec((NC, BP, H), lambda i: (0, i, 0)),
        pl.BlockSpec((NC, BP, 16), lambda i: (0, i, 0)),
        pl.BlockSpec((BP, 1), lambda i: (i, 0)),
        pl.BlockSpec((H, H), lambda i: (0, 0)),
        pl.BlockSpec((H, H), lambda i: (0, 0)),
        pl.BlockSpec((1, H), lambda i: (0, 0)),
        pl.BlockSpec((1, H), lambda i: (0, 0)),
        pl.BlockSpec((1, H), lambda i: (0, 0)),
        pl.BlockSpec((G, H), lambda i: (0, 0)),
    ],
    out_specs=[
        pl.BlockSpec((BP, H), lambda i: (i, 0)),
        pl.BlockSpec((G, H), lambda i: (0, 0)),
    ],
    out_shape=[
        jax.ShapeDtypeStruct((N, H), jnp.float32),
        jax.ShapeDtypeStruct((G, H), jnp.float32),
    ],
)


def kernel(x, edge_index, batch, fc_W, fc_b, W_self, W_neigh, b_conv, gamma, beta):
    src = edge_index[0].astype(jnp.int32)
    dst = edge_index[1].astype(jnp.int32)
    pad = E_PAD - E
    pad_src = jnp.arange(pad, dtype=jnp.int32) % N
    src_p = jnp.concatenate([src, pad_src]).reshape(NW, NJ, ROWS)
    pad_dst = DUMMY + jnp.arange(pad, dtype=jnp.int32) % (N_ACC - N)
    dst_p = jnp.concatenate([dst, pad_dst]).reshape(NW, NJ, ROWS)
    batch_r = batch.astype(jnp.int32).reshape(N, 1)

    zeros16 = jnp.zeros((N_ACC, 16), jnp.float32)
    zeros64 = jnp.zeros((N_ACC, H), jnp.float32)
    ones16 = jnp.ones((ROWS, 16), jnp.float32)

    deg_parts = _deg_call(dst_p, ones16, zeros16)
    h = _proj_call(x, fc_W, fc_b.reshape(1, H))
    ge = jnp.zeros((G, H), jnp.float32)
    for i in range(NB):
        agg_parts = _agg_call(h, src_p, dst_p, zeros64)
        h, ge = _layer_call(h, agg_parts, deg_parts, batch_r,
                            W_self[i], W_neigh[i],
                            b_conv[i].reshape(1, H),
                            gamma[i].reshape(1, H),
                            beta[i].reshape(1, H), ge)
    return (h, ge)
